# Initial kernel scaffold; baseline (speedup 1.0000x reference)
import jax
import jax.numpy as jnp
from jax.experimental import pallas as pl


def kernel(x, h, edge_attr, e_w, xk_w1, xk_b1, xk_g, xk_be, xk_w2, xk_b2, xv_w1, xv_b1, xv_g, xv_be, xv_w2, xv_b2, xq_w1, xq_b1, xq_g, xq_be, xq_w2, xq_b2, edge_index):
    raise NotImplementedError("write your pallas kernel here")


# speedup vs baseline: 12.3221x; 12.3221x over previous
import functools
import math

import jax
import jax.numpy as jnp
from jax import lax
from jax.experimental import pallas as pl
from jax.experimental.pallas import tpu as pltpu
from jax.experimental.pallas import tpu_sc as plsc

_N = 10000
_E = 320000
_IN = 128
_HID = 128
_OUT = 128
_NH = 16
_DH = 8
_ED = 4
_NG = 20
_RMAX = 10.0

_NC = 2
_NS = 16
_NW = _NC * _NS
_EPW = _E // _NW
_CH = 80
_NCHUNK = _EPW // _CH
_NPAD = 10240
_ROWS_PER_TILE = _NPAD // _NS

_DST_W = _IN + _OUT + 16
_SRC_W = _IN + 16


def _mesh():
    return plsc.VectorSubcoreMesh(core_axis_name="c", subcore_axis_name="s")


def _sc_gather2_body(tdst_h, tsrc_h, di_h, si_h, od_h, os_h,
                     di_v, si_v, db_v, sb_v, sem1, sem2):
    c = lax.axis_index("c")
    s = lax.axis_index("s")
    ebase = (c * _NS + s) * _EPW

    def step(i, carry):
        base = pl.multiple_of(ebase + i * _CH, 8)
        pltpu.sync_copy(di_h.at[pl.ds(base, _CH)], di_v)
        pltpu.sync_copy(si_h.at[pl.ds(base, _CH)], si_v)
        cp1 = pltpu.async_copy(tdst_h.at[di_v], db_v, sem1)
        cp2 = pltpu.async_copy(tsrc_h.at[si_v], sb_v, sem2)
        cp1.wait()
        cp2.wait()
        pltpu.sync_copy(db_v, od_h.at[pl.ds(base, _CH)])
        pltpu.sync_copy(sb_v, os_h.at[pl.ds(base, _CH)])
        return carry

    lax.fori_loop(0, _NCHUNK, step, 0)


def _sc_gather2(tdst, tsrc, dsti, srci):
    return pl.kernel(
        _sc_gather2_body,
        out_type=[jax.ShapeDtypeStruct((_E, _DST_W), jnp.float32),
                  jax.ShapeDtypeStruct((_E, _SRC_W), jnp.float32)],
        mesh=_mesh(),
        compiler_params=pltpu.CompilerParams(use_tc_tiling_on_sc=False),
        scratch_types=[pltpu.VMEM((_CH,), jnp.int32),
                       pltpu.VMEM((_CH,), jnp.int32),
                       pltpu.VMEM((_CH, _DST_W), jnp.float32),
                       pltpu.VMEM((_CH, _SRC_W), jnp.float32),
                       pltpu.SemaphoreType.DMA,
                       pltpu.SemaphoreType.DMA],
    )(tdst, tsrc, dsti, srci)


def _sc_gather1_body(tab_h, di_h, o_h, di_v, b_v, sem):
    c = lax.axis_index("c")
    s = lax.axis_index("s")
    ebase = (c * _NS + s) * _EPW

    def step(i, carry):
        base = pl.multiple_of(ebase + i * _CH, 8)
        pltpu.sync_copy(di_h.at[pl.ds(base, _CH)], di_v)
        pltpu.async_copy(tab_h.at[di_v], b_v, sem).wait()
        pltpu.sync_copy(b_v, o_h.at[pl.ds(base, _CH)])
        return carry

    lax.fori_loop(0, _NCHUNK, step, 0)


def _sc_gather1(tab, dsti, width):
    return pl.kernel(
        _sc_gather1_body,
        out_type=jax.ShapeDtypeStruct((_E, width), jnp.float32),
        mesh=_mesh(),
        compiler_params=pltpu.CompilerParams(use_tc_tiling_on_sc=False),
        scratch_types=[pltpu.VMEM((_CH,), jnp.int32),
                       pltpu.VMEM((_CH, width), jnp.float32),
                       pltpu.SemaphoreType.DMA],
    )(tab, dsti)


def _sc_scatter_body(d_h, di_h, z_h, out_h, di_v, d_v, shared):
    c = lax.axis_index("c")
    s = lax.axis_index("s")
    rbase = pl.multiple_of(s * _ROWS_PER_TILE, 8)
    pltpu.sync_copy(z_h.at[pl.ds(rbase, _ROWS_PER_TILE)],
                    shared.at[pl.ds(rbase, _ROWS_PER_TILE)])
    plsc.subcore_barrier()
    ebase = (c * _NS + s) * _EPW

    def step(i, carry):
        base = pl.multiple_of(ebase + i * _CH, 8)
        pltpu.sync_copy(di_h.at[pl.ds(base, _CH)], di_v)
        pltpu.sync_copy(d_h.at[pl.ds(base, _CH)], d_v)
        pltpu.sync_copy(d_v, shared.at[di_v], add=True)
        return carry

    lax.fori_loop(0, _NCHUNK, step, 0)
    plsc.subcore_barrier()
    pltpu.sync_copy(shared.at[pl.ds(rbase, _ROWS_PER_TILE)],
                    out_h.at[c, pl.ds(rbase, _ROWS_PER_TILE)])


def _sc_scatter(data, dsti, width):
    zeros = jnp.zeros((_NPAD, width), jnp.float32)
    return pl.kernel(
        _sc_scatter_body,
        out_type=jax.ShapeDtypeStruct((_NC, _NPAD, width), jnp.float32),
        mesh=_mesh(),
        compiler_params=pltpu.CompilerParams(use_tc_tiling_on_sc=False),
        scratch_types=[pltpu.VMEM((_CH,), jnp.int32),
                       pltpu.VMEM((_CH, width), jnp.float32),
                       pltpu.VMEM_SHARED((_NPAD, width), jnp.float32)],
    )(data, dsti, zeros)


def _q_body(h_r, w1, b1, g, be, w2, b2, o_r):
    hdn = jnp.dot(h_r[...], w1[...], preferred_element_type=jnp.float32) + b1[...]
    mu = jnp.mean(hdn, -1, keepdims=True)
    var = jnp.mean(jnp.square(hdn - mu), -1, keepdims=True)
    hdn = (hdn - mu) * lax.rsqrt(var + 1e-5) * g[...] + be[...]
    hdn = jnp.maximum(hdn, 0.0)
    o_r[...] = jnp.dot(hdn, w2[...], preferred_element_type=jnp.float32) + b2[...]


def _qmlp(h, w1, b1, g, be, w2, b2):
    blk = 2000
    grid = _N // blk
    full = lambda shape: pl.BlockSpec(shape, lambda i: (0, 0))
    return pl.pallas_call(
        _q_body,
        grid=(grid,),
        in_specs=[pl.BlockSpec((blk, _IN), lambda i: (i, 0)),
                  full((_IN, _HID)), full((1, _HID)), full((1, _HID)),
                  full((1, _HID)), full((_HID, _OUT)), full((1, _OUT))],
        out_specs=pl.BlockSpec((blk, _OUT), lambda i: (i, 0)),
        out_shape=jax.ShapeDtypeStruct((_N, _OUT), jnp.float32),
    )(h, w1, b1, g, be, w2, b2)


_BE = 2000


def _edge_body(dst_r, src_r, ea_r, ew_r,
               wk1a, wk1b, wk1cd, bk1, gk, bek, wk2, bk2,
               wv1a, wv1b, wv1cd, bv1, gv, bev, wv2, bv2,
               logits_r, v3_r, gmax_r):
    hi = dst_r[:, 0:_IN]
    qd = dst_r[:, _IN:_IN + _OUT]
    xd = dst_r[:, _IN + _OUT:_DST_W]
    hj = src_r[:, 0:_IN]
    xs = src_r[:, _IN:_SRC_W]

    rel = xd - xs
    dist = jnp.sqrt(jnp.sum(rel * rel, -1, keepdims=True))
    spacing = _RMAX / (_NG - 1)
    coeff = -0.5 / (spacing * spacing)
    offs = lax.broadcasted_iota(jnp.int32, (1, _NG), 1).astype(jnp.float32) * spacing
    dist_feat = jnp.exp(coeff * jnp.square(dist - offs))

    ea = ea_r[...]
    df = jnp.concatenate([ea[:, a:a + 1] * dist_feat for a in range(_ED)], -1)
    hh = jnp.concatenate([hi, hj], -1)

    def front(w1a, w1b, w1cd, b1, g, be):
        pre = (jnp.dot(ea, w1a[...], preferred_element_type=jnp.float32)
               + jnp.dot(df, w1b[...], preferred_element_type=jnp.float32)
               + jnp.dot(hh, w1cd[...], preferred_element_type=jnp.float32)
               + b1[...])
        mu = jnp.mean(pre, -1, keepdims=True)
        var = jnp.mean(jnp.square(pre - mu), -1, keepdims=True)
        pre = (pre - mu) * lax.rsqrt(var + 1e-5) * g[...] + be[...]
        return jnp.maximum(pre, 0.0)

    hk = front(wk1a, wk1b, wk1cd, bk1, gk, bek)
    k = jnp.dot(hk, wk2[...], preferred_element_type=jnp.float32) + bk2[...]
    hv = front(wv1a, wv1b, wv1cd, bv1, gv, bev)
    v = jnp.dot(hv, wv2[...], preferred_element_type=jnp.float32) + bv2[...]
    v = v * ew_r[...]

    sel = (lax.broadcasted_iota(jnp.int32, (_OUT, _NH), 0) // _DH
           == lax.broadcasted_iota(jnp.int32, (_OUT, _NH), 1)).astype(jnp.float32)
    logits = jnp.dot(k * qd, sel, preferred_element_type=jnp.float32)
    logits = logits * (1.0 / math.sqrt(_DH))
    logits_r[...] = logits

    v3_r[...] = jnp.concatenate(
        [v * rel[:, 0:1], v * rel[:, 1:2], v * rel[:, 2:3]], -1)

    @pl.when(pl.program_id(0) == 0)
    def _():
        gmax_r[...] = jnp.full((8, 128), -jnp.inf, jnp.float32)

    cur = gmax_r[0:1, 0:_NH]
    gmax_r[0:1, 0:_NH] = jnp.maximum(cur, jnp.max(logits, 0, keepdims=True))


def _edge_tc(dstrows, srcrows, ea, ew, wk, wv):
    grid = _E // _BE
    full = lambda shape: pl.BlockSpec(shape, lambda i: (0, 0))
    wspecs = [full((_ED, _HID)), full((_ED * _NG, _HID)), full((2 * _IN, _HID)),
              full((1, _HID)), full((1, _HID)), full((1, _HID))]
    return pl.pallas_call(
        _edge_body,
        grid=(grid,),
        in_specs=([pl.BlockSpec((_BE, _DST_W), lambda i: (i, 0)),
                   pl.BlockSpec((_BE, _SRC_W), lambda i: (i, 0)),
                   pl.BlockSpec((_BE, _ED), lambda i: (i, 0)),
                   pl.BlockSpec((_BE, 1), lambda i: (i, 0))]
                  + wspecs + [full((_HID, _OUT)), full((1, _OUT))]
                  + wspecs + [full((_HID, _NH)), full((1, _NH))]),
        out_specs=[pl.BlockSpec((_BE, _NH), lambda i: (i, 0)),
                   pl.BlockSpec((_BE, 3 * _NH), lambda i: (i, 0)),
                   pl.BlockSpec((8, 128), lambda i: (0, 0))],
        out_shape=[jax.ShapeDtypeStruct((_E, _NH), jnp.float32),
                   jax.ShapeDtypeStruct((_E, 3 * _NH), jnp.float32),
                   jax.ShapeDtypeStruct((8, 128), jnp.float32)],
    )(dstrows, srcrows, ea, ew, *wk, *wv)


def _exp_body(l_r, gmax_r, o_r):
    o_r[...] = jnp.exp(l_r[...] - gmax_r[0:1, 0:_NH])


def _exp_tc(logits, gmax):
    blk = 8000
    return pl.pallas_call(
        _exp_body,
        grid=(_E // blk,),
        in_specs=[pl.BlockSpec((blk, _NH), lambda i: (i, 0)),
                  pl.BlockSpec((8, 128), lambda i: (0, 0))],
        out_specs=pl.BlockSpec((blk, _NH), lambda i: (i, 0)),
        out_shape=jax.ShapeDtypeStruct((_E, _NH), jnp.float32),
    )(logits, gmax)


def _av_body(ex_r, dg_r, v3_r, o_r):
    al = ex_r[...] / (dg_r[...] + 1e-16)
    v3 = v3_r[...]
    o_r[...] = jnp.concatenate(
        [al * v3[:, 0:_NH], al * v3[:, _NH:2 * _NH], al * v3[:, 2 * _NH:3 * _NH]], -1)


def _av_tc(ex, dg, v3):
    blk = 8000
    return pl.pallas_call(
        _av_body,
        grid=(_E // blk,),
        in_specs=[pl.BlockSpec((blk, _NH), lambda i: (i, 0)),
                  pl.BlockSpec((blk, _NH), lambda i: (i, 0)),
                  pl.BlockSpec((blk, 3 * _NH), lambda i: (i, 0))],
        out_specs=pl.BlockSpec((blk, 3 * _NH), lambda i: (i, 0)),
        out_shape=jax.ShapeDtypeStruct((_E, 3 * _NH), jnp.float32),
    )(ex, dg, v3)


def kernel(x, h, edge_attr, e_w, xk_w1, xk_b1, xk_g, xk_be, xk_w2, xk_b2,
           xv_w1, xv_b1, xv_g, xv_be, xv_w2, xv_b2,
           xq_w1, xq_b1, xq_g, xq_be, xq_w2, xq_b2, edge_index):
    src = edge_index[0]
    dst = edge_index[1]
    row = lambda a: a.reshape(1, -1)

    q = _qmlp(h, xq_w1, row(xq_b1), row(xq_g), row(xq_be), xq_w2, row(xq_b2))

    xpad = jnp.pad(x, ((0, 0), (0, 13)))
    tdst = jnp.concatenate([h, q, xpad], 1)
    tsrc = jnp.concatenate([h, xpad], 1)
    dstrows, srcrows = _sc_gather2(tdst, tsrc, dst, src)

    def split(w1):
        return w1[0:_ED], w1[_ED:_ED + _ED * _NG], w1[_ED + _ED * _NG:]

    ka, kb, kcd = split(xk_w1)
    va, vb, vcd = split(xv_w1)
    wk = (ka, kb, kcd, row(xk_b1), row(xk_g), row(xk_be), xk_w2, row(xk_b2))
    wv = (va, vb, vcd, row(xv_b1), row(xv_g), row(xv_be), xv_w2, row(xv_b2))

    logits, v3, gmax = _edge_tc(dstrows, srcrows, edge_attr, e_w, wk, wv)
    ex = _exp_tc(logits, gmax)

    dpart = _sc_scatter(ex, dst, _NH)
    denom = dpart[0] + dpart[1]
    dg = _sc_gather1(denom, dst, _NH)

    m3 = _av_tc(ex, dg, v3)
    opart = _sc_scatter(m3, dst, 3 * _NH)
    out = (opart[0, :_N] + opart[1, :_N]).reshape(_N, 3, _NH)
    return jnp.mean(out, -1)


# speedup vs baseline: 13.9492x; 1.1320x over previous
import functools
import math

import jax
import jax.numpy as jnp
from jax import lax
from jax.experimental import pallas as pl
from jax.experimental.pallas import tpu as pltpu
from jax.experimental.pallas import tpu_sc as plsc

_N = 10000
_E = 320000
_IN = 128
_HID = 128
_OUT = 128
_NH = 16
_DH = 8
_ED = 4
_NG = 20
_RMAX = 10.0

_NC = 2
_NS = 16
_NW = _NC * _NS
_EPW = _E // _NW
_CH = 80
_NCHUNK = _EPW // _CH
_NPAD = 10240
_ROWS_PER_TILE = _NPAD // _NS

_DST_W = _IN + _OUT + 16
_SRC_W = _IN + 16


def _mesh():
    return plsc.VectorSubcoreMesh(core_axis_name="c", subcore_axis_name="s")


def _sc_gather2_body(tdst_h, tsrc_h, di_h, si_h, od_h, os_h,
                     di_v, si_v, db_v, sb_v, sem1, sem2):
    c = lax.axis_index("c")
    s = lax.axis_index("s")
    ebase = (c * _NS + s) * _EPW

    def step(i, carry):
        base = pl.multiple_of(ebase + i * _CH, 8)
        pltpu.sync_copy(di_h.at[pl.ds(base, _CH)], di_v)
        pltpu.sync_copy(si_h.at[pl.ds(base, _CH)], si_v)
        cp1 = pltpu.async_copy(tdst_h.at[di_v], db_v, sem1)
        cp2 = pltpu.async_copy(tsrc_h.at[si_v], sb_v, sem2)
        cp1.wait()
        cp2.wait()
        pltpu.sync_copy(db_v, od_h.at[pl.ds(base, _CH)])
        pltpu.sync_copy(sb_v, os_h.at[pl.ds(base, _CH)])
        return carry

    lax.fori_loop(0, _NCHUNK, step, 0)


def _sc_gather2(tdst, tsrc, dsti, srci):
    return pl.kernel(
        _sc_gather2_body,
        out_type=[jax.ShapeDtypeStruct((_E, _DST_W), jnp.float32),
                  jax.ShapeDtypeStruct((_E, _SRC_W), jnp.float32)],
        mesh=_mesh(),
        compiler_params=pltpu.CompilerParams(use_tc_tiling_on_sc=False),
        scratch_types=[pltpu.VMEM((_CH,), jnp.int32),
                       pltpu.VMEM((_CH,), jnp.int32),
                       pltpu.VMEM((_CH, _DST_W), jnp.float32),
                       pltpu.VMEM((_CH, _SRC_W), jnp.float32),
                       pltpu.SemaphoreType.DMA,
                       pltpu.SemaphoreType.DMA],
    )(tdst, tsrc, dsti, srci)


def _sc_gather1_body(tab_h, di_h, o_h, di_v, b_v, sem):
    c = lax.axis_index("c")
    s = lax.axis_index("s")
    ebase = (c * _NS + s) * _EPW

    def step(i, carry):
        base = pl.multiple_of(ebase + i * _CH, 8)
        pltpu.sync_copy(di_h.at[pl.ds(base, _CH)], di_v)
        pltpu.async_copy(tab_h.at[di_v], b_v, sem).wait()
        pltpu.sync_copy(b_v, o_h.at[pl.ds(base, _CH)])
        return carry

    lax.fori_loop(0, _NCHUNK, step, 0)


def _sc_gather1(tab, dsti, width):
    return pl.kernel(
        _sc_gather1_body,
        out_type=jax.ShapeDtypeStruct((_E, width), jnp.float32),
        mesh=_mesh(),
        compiler_params=pltpu.CompilerParams(use_tc_tiling_on_sc=False),
        scratch_types=[pltpu.VMEM((_CH,), jnp.int32),
                       pltpu.VMEM((_CH, width), jnp.float32),
                       pltpu.SemaphoreType.DMA],
    )(tab, dsti)


def _sc_scatter_body(d_h, di_h, z_h, out_h, di_v, d_v, shared):
    c = lax.axis_index("c")
    s = lax.axis_index("s")
    rbase = pl.multiple_of(s * _ROWS_PER_TILE, 8)
    pltpu.sync_copy(z_h.at[pl.ds(rbase, _ROWS_PER_TILE)],
                    shared.at[pl.ds(rbase, _ROWS_PER_TILE)])
    plsc.subcore_barrier()
    ebase = (c * _NS + s) * _EPW

    def step(i, carry):
        base = pl.multiple_of(ebase + i * _CH, 8)
        pltpu.sync_copy(di_h.at[pl.ds(base, _CH)], di_v)
        pltpu.sync_copy(d_h.at[pl.ds(base, _CH)], d_v)
        pltpu.sync_copy(d_v, shared.at[di_v], add=True)
        return carry

    lax.fori_loop(0, _NCHUNK, step, 0)
    plsc.subcore_barrier()
    pltpu.sync_copy(shared.at[pl.ds(rbase, _ROWS_PER_TILE)],
                    out_h.at[c, pl.ds(rbase, _ROWS_PER_TILE)])


def _sc_scatter(data, dsti, width):
    zeros = jnp.zeros((_NPAD, width), jnp.float32)
    return pl.kernel(
        _sc_scatter_body,
        out_type=jax.ShapeDtypeStruct((_NC, _NPAD, width), jnp.float32),
        mesh=_mesh(),
        compiler_params=pltpu.CompilerParams(use_tc_tiling_on_sc=False),
        scratch_types=[pltpu.VMEM((_CH,), jnp.int32),
                       pltpu.VMEM((_CH, width), jnp.float32),
                       pltpu.VMEM_SHARED((_NPAD, width), jnp.float32)],
    )(data, dsti, zeros)


def _q_body(h_r, x_r, w1, b1, g, be, w2, b2, od_r, os_r):
    hh = h_r[...]
    hdn = jnp.dot(hh, w1[...], preferred_element_type=jnp.float32) + b1[...]
    mu = jnp.mean(hdn, -1, keepdims=True)
    var = jnp.mean(jnp.square(hdn - mu), -1, keepdims=True)
    hdn = (hdn - mu) * lax.rsqrt(var + 1e-5) * g[...] + be[...]
    hdn = jnp.maximum(hdn, 0.0)
    q = jnp.dot(hdn, w2[...], preferred_element_type=jnp.float32) + b2[...]
    blk = hh.shape[0]
    xp = jnp.concatenate([x_r[...], jnp.zeros((blk, 13), jnp.float32)], -1)
    od_r[:, 0:_IN] = hh
    od_r[:, _IN:_IN + _OUT] = q
    od_r[:, _IN + _OUT:_DST_W] = xp
    os_r[:, 0:_IN] = hh
    os_r[:, _IN:_SRC_W] = xp


def _qmlp(h, x, w1, b1, g, be, w2, b2):
    blk = 2000
    grid = _N // blk
    full = lambda shape: pl.BlockSpec(shape, lambda i: (0, 0))
    return pl.pallas_call(
        _q_body,
        grid=(grid,),
        in_specs=[pl.BlockSpec((blk, _IN), lambda i: (i, 0)),
                  pl.BlockSpec((blk, 3), lambda i: (i, 0)),
                  full((_IN, _HID)), full((1, _HID)), full((1, _HID)),
                  full((1, _HID)), full((_HID, _OUT)), full((1, _OUT))],
        out_specs=[pl.BlockSpec((blk, _DST_W), lambda i: (i, 0)),
                   pl.BlockSpec((blk, _SRC_W), lambda i: (i, 0))],
        out_shape=[jax.ShapeDtypeStruct((_N, _DST_W), jnp.float32),
                   jax.ShapeDtypeStruct((_N, _SRC_W), jnp.float32)],
    )(h, x, w1, b1, g, be, w2, b2)


_BE = 2000


def _edge_body(dst_r, src_r, ea_r, ew_r,
               wk1a, wk1b, wk1cd, bk1, gk, bek, wk2, bk2,
               wv1a, wv1b, wv1cd, bv1, gv, bev, wv2, bv2,
               logits_r, v3_r, gmax_r):
    hi = dst_r[:, 0:_IN]
    qd = dst_r[:, _IN:_IN + _OUT]
    xd = dst_r[:, _IN + _OUT:_DST_W]
    hj = src_r[:, 0:_IN]
    xs = src_r[:, _IN:_SRC_W]

    rel = xd - xs
    spacing = _RMAX / (_NG - 1)
    coeff = -0.5 / (spacing * spacing)
    ones_b = jnp.ones((16, _NG), jnp.float32)
    distb = jnp.sqrt(jnp.dot(rel * rel, ones_b, preferred_element_type=jnp.float32))
    offs = lax.broadcasted_iota(jnp.int32, (1, _NG), 1).astype(jnp.float32) * spacing
    dist_feat = jnp.exp(coeff * jnp.square(distb - offs))

    ea = ea_r[...]
    i0 = lambda s: lax.broadcasted_iota(jnp.int32, s, 0)
    i1 = lambda s: lax.broadcasted_iota(jnp.int32, s, 1)
    exp_ea = (i1((_ED, _ED * _NG)) // _NG == i0((_ED, _ED * _NG))).astype(jnp.float32)
    til_df = (i1((_NG, _ED * _NG)) % _NG == i0((_NG, _ED * _NG))).astype(jnp.float32)
    df = (jnp.dot(ea, exp_ea, preferred_element_type=jnp.float32)
          * jnp.dot(dist_feat, til_df, preferred_element_type=jnp.float32))
    hh = jnp.concatenate([hi, hj], -1)

    def front(w1a, w1b, w1cd, b1, g, be):
        pre = (jnp.dot(ea, w1a[...], preferred_element_type=jnp.float32)
               + jnp.dot(df, w1b[...], preferred_element_type=jnp.float32)
               + jnp.dot(hh, w1cd[...], preferred_element_type=jnp.float32)
               + b1[...])
        mu = jnp.mean(pre, -1, keepdims=True)
        var = jnp.mean(jnp.square(pre - mu), -1, keepdims=True)
        pre = (pre - mu) * lax.rsqrt(var + 1e-5) * g[...] + be[...]
        return jnp.maximum(pre, 0.0)

    hk = front(wk1a, wk1b, wk1cd, bk1, gk, bek)
    k = jnp.dot(hk, wk2[...], preferred_element_type=jnp.float32) + bk2[...]
    hv = front(wv1a, wv1b, wv1cd, bv1, gv, bev)
    v = jnp.dot(hv, wv2[...], preferred_element_type=jnp.float32) + bv2[...]
    v = v * ew_r[...]

    sel = (lax.broadcasted_iota(jnp.int32, (_OUT, _NH), 0) // _DH
           == lax.broadcasted_iota(jnp.int32, (_OUT, _NH), 1)).astype(jnp.float32)
    logits = jnp.dot(k * qd, sel, preferred_element_type=jnp.float32)
    logits = logits * (1.0 / math.sqrt(_DH))
    logits_r[...] = logits

    til_v = (i1((_NH, 3 * _NH)) % _NH == i0((_NH, 3 * _NH))).astype(jnp.float32)
    exp_r = (i1((16, 3 * _NH)) // _NH == i0((16, 3 * _NH))).astype(jnp.float32)
    v3_r[...] = (jnp.dot(v, til_v, preferred_element_type=jnp.float32)
                 * jnp.dot(rel, exp_r, preferred_element_type=jnp.float32))

    @pl.when(pl.program_id(0) == 0)
    def _():
        gmax_r[...] = jnp.full((8, 128), -jnp.inf, jnp.float32)

    cur = gmax_r[0:1, 0:_NH]
    gmax_r[0:1, 0:_NH] = jnp.maximum(cur, jnp.max(logits, 0, keepdims=True))


def _edge_tc(dstrows, srcrows, ea, ew, wk, wv):
    grid = _E // _BE
    full = lambda shape: pl.BlockSpec(shape, lambda i: (0, 0))
    wspecs = [full((_ED, _HID)), full((_ED * _NG, _HID)), full((2 * _IN, _HID)),
              full((1, _HID)), full((1, _HID)), full((1, _HID))]
    return pl.pallas_call(
        _edge_body,
        grid=(grid,),
        in_specs=([pl.BlockSpec((_BE, _DST_W), lambda i: (i, 0)),
                   pl.BlockSpec((_BE, _SRC_W), lambda i: (i, 0)),
                   pl.BlockSpec((_BE, _ED), lambda i: (i, 0)),
                   pl.BlockSpec((_BE, 1), lambda i: (i, 0))]
                  + wspecs + [full((_HID, _OUT)), full((1, _OUT))]
                  + wspecs + [full((_HID, _NH)), full((1, _NH))]),
        out_specs=[pl.BlockSpec((_BE, _NH), lambda i: (i, 0)),
                   pl.BlockSpec((_BE, 3 * _NH), lambda i: (i, 0)),
                   pl.BlockSpec((8, 128), lambda i: (0, 0))],
        out_shape=[jax.ShapeDtypeStruct((_E, _NH), jnp.float32),
                   jax.ShapeDtypeStruct((_E, 3 * _NH), jnp.float32),
                   jax.ShapeDtypeStruct((8, 128), jnp.float32)],
    )(dstrows, srcrows, ea, ew, *wk, *wv)


def _exp_body(l_r, gmax_r, o_r):
    o_r[...] = jnp.exp(l_r[...] - gmax_r[0:1, 0:_NH])


def _exp_tc(logits, gmax):
    blk = 8000
    return pl.pallas_call(
        _exp_body,
        grid=(_E // blk,),
        in_specs=[pl.BlockSpec((blk, _NH), lambda i: (i, 0)),
                  pl.BlockSpec((8, 128), lambda i: (0, 0))],
        out_specs=pl.BlockSpec((blk, _NH), lambda i: (i, 0)),
        out_shape=jax.ShapeDtypeStruct((_E, _NH), jnp.float32),
    )(logits, gmax)


def _av_body(ex_r, dg_r, v3_r, o_r):
    al = ex_r[...] / (dg_r[...] + 1e-16)
    i0 = lambda s: lax.broadcasted_iota(jnp.int32, s, 0)
    i1 = lambda s: lax.broadcasted_iota(jnp.int32, s, 1)
    til = (i1((_NH, 3 * _NH)) % _NH == i0((_NH, 3 * _NH))).astype(jnp.float32)
    o_r[...] = jnp.dot(al, til, preferred_element_type=jnp.float32) * v3_r[...]


def _av_tc(ex, dg, v3):
    blk = 8000
    return pl.pallas_call(
        _av_body,
        grid=(_E // blk,),
        in_specs=[pl.BlockSpec((blk, _NH), lambda i: (i, 0)),
                  pl.BlockSpec((blk, _NH), lambda i: (i, 0)),
                  pl.BlockSpec((blk, 3 * _NH), lambda i: (i, 0))],
        out_specs=pl.BlockSpec((blk, 3 * _NH), lambda i: (i, 0)),
        out_shape=jax.ShapeDtypeStruct((_E, 3 * _NH), jnp.float32),
    )(ex, dg, v3)


def kernel(x, h, edge_attr, e_w, xk_w1, xk_b1, xk_g, xk_be, xk_w2, xk_b2,
           xv_w1, xv_b1, xv_g, xv_be, xv_w2, xv_b2,
           xq_w1, xq_b1, xq_g, xq_be, xq_w2, xq_b2, edge_index):
    src = edge_index[0]
    dst = edge_index[1]
    row = lambda a: a.reshape(1, -1)

    tdst, tsrc = _qmlp(h, x, xq_w1, row(xq_b1), row(xq_g), row(xq_be),
                       xq_w2, row(xq_b2))
    dstrows, srcrows = _sc_gather2(tdst, tsrc, dst, src)

    def split(w1):
        return w1[0:_ED], w1[_ED:_ED + _ED * _NG], w1[_ED + _ED * _NG:]

    ka, kb, kcd = split(xk_w1)
    va, vb, vcd = split(xv_w1)
    wk = (ka, kb, kcd, row(xk_b1), row(xk_g), row(xk_be), xk_w2, row(xk_b2))
    wv = (va, vb, vcd, row(xv_b1), row(xv_g), row(xv_be), xv_w2, row(xv_b2))

    logits, v3, gmax = _edge_tc(dstrows, srcrows, edge_attr, e_w, wk, wv)
    ex = _exp_tc(logits, gmax)

    dpart = _sc_scatter(ex, dst, _NH)
    denom = dpart[0] + dpart[1]
    dg = _sc_gather1(denom, dst, _NH)

    m3 = _av_tc(ex, dg, v3)
    opart = _sc_scatter(m3, dst, 3 * _NH)
    out = (opart[0, :_N] + opart[1, :_N]).reshape(_N, 3, _NH)
    return jnp.mean(out, -1)


# speedup vs baseline: 17.4509x; 1.2510x over previous
import functools
import math

import jax
import jax.numpy as jnp
from jax import lax
from jax.experimental import pallas as pl
from jax.experimental.pallas import tpu as pltpu
from jax.experimental.pallas import tpu_sc as plsc

_N = 10000
_E = 320000
_IN = 128
_HID = 128
_OUT = 128
_NH = 16
_DH = 8
_ED = 4
_NG = 20
_RMAX = 10.0

_NC = 2
_NS = 16
_NW = _NC * _NS
_EPW = _E // _NW
_CH = 80
_NCHUNK = _EPW // _CH
_NPAD = 10240
_ROWS_PER_TILE = _NPAD // _NS

_DST_W = _IN + _OUT + 16
_SRC_W = _IN + 16


def _mesh():
    return plsc.VectorSubcoreMesh(core_axis_name="c", subcore_axis_name="s")


def _sc_gather5_body(th_h, tq_h, tx_h, di_h, si_h,
                     hd_h, qd_h, hs_h, xd_h, xs_h,
                     di_v, si_v, bhd, bqd, bhs, bxd, bxs,
                     s1, s2, s3, s4, s5):
    c = lax.axis_index("c")
    s = lax.axis_index("s")
    ebase = (c * _NS + s) * _EPW

    def step(i, carry):
        base = pl.multiple_of(ebase + i * _CH, 8)
        pltpu.sync_copy(di_h.at[pl.ds(base, _CH)], di_v)
        pltpu.sync_copy(si_h.at[pl.ds(base, _CH)], si_v)
        cps = [pltpu.async_copy(th_h.at[di_v], bhd, s1),
               pltpu.async_copy(tq_h.at[di_v], bqd, s2),
               pltpu.async_copy(th_h.at[si_v], bhs, s3),
               pltpu.async_copy(tx_h.at[di_v], bxd, s4),
               pltpu.async_copy(tx_h.at[si_v], bxs, s5)]
        for cp in cps:
            cp.wait()
        pltpu.sync_copy(bhd, hd_h.at[pl.ds(base, _CH)])
        pltpu.sync_copy(bqd, qd_h.at[pl.ds(base, _CH)])
        pltpu.sync_copy(bhs, hs_h.at[pl.ds(base, _CH)])
        pltpu.sync_copy(bxd, xd_h.at[pl.ds(base, _CH)])
        pltpu.sync_copy(bxs, xs_h.at[pl.ds(base, _CH)])
        return carry

    lax.fori_loop(0, _NCHUNK, step, 0)


def _sc_gather5(th, tq, tx, dsti, srci):
    f32 = jnp.float32
    return pl.kernel(
        _sc_gather5_body,
        out_type=[jax.ShapeDtypeStruct((_E, 128), f32),
                  jax.ShapeDtypeStruct((_E, 128), f32),
                  jax.ShapeDtypeStruct((_E, 128), f32),
                  jax.ShapeDtypeStruct((_E, 16), f32),
                  jax.ShapeDtypeStruct((_E, 16), f32)],
        mesh=_mesh(),
        compiler_params=pltpu.CompilerParams(use_tc_tiling_on_sc=False),
        scratch_types=[pltpu.VMEM((_CH,), jnp.int32),
                       pltpu.VMEM((_CH,), jnp.int32),
                       pltpu.VMEM((_CH, 128), f32),
                       pltpu.VMEM((_CH, 128), f32),
                       pltpu.VMEM((_CH, 128), f32),
                       pltpu.VMEM((_CH, 16), f32),
                       pltpu.VMEM((_CH, 16), f32),
                       pltpu.SemaphoreType.DMA,
                       pltpu.SemaphoreType.DMA,
                       pltpu.SemaphoreType.DMA,
                       pltpu.SemaphoreType.DMA,
                       pltpu.SemaphoreType.DMA],
    )(th, tq, tx, dsti, srci)


def _sc_gather1_body(tab_h, di_h, o_h, di_v, b_v, sem):
    c = lax.axis_index("c")
    s = lax.axis_index("s")
    ebase = (c * _NS + s) * _EPW

    def step(i, carry):
        base = pl.multiple_of(ebase + i * _CH, 8)
        pltpu.sync_copy(di_h.at[pl.ds(base, _CH)], di_v)
        pltpu.async_copy(tab_h.at[di_v], b_v, sem).wait()
        pltpu.sync_copy(b_v, o_h.at[pl.ds(base, _CH)])
        return carry

    lax.fori_loop(0, _NCHUNK, step, 0)


def _sc_gather1(tab, dsti, width):
    return pl.kernel(
        _sc_gather1_body,
        out_type=jax.ShapeDtypeStruct((_E, width), jnp.float32),
        mesh=_mesh(),
        compiler_params=pltpu.CompilerParams(use_tc_tiling_on_sc=False),
        scratch_types=[pltpu.VMEM((_CH,), jnp.int32),
                       pltpu.VMEM((_CH, width), jnp.float32),
                       pltpu.SemaphoreType.DMA],
    )(tab, dsti)


def _sc_scatter_body(d_h, di_h, z_h, out_h, di_v, d_v, shared):
    c = lax.axis_index("c")
    s = lax.axis_index("s")
    rbase = pl.multiple_of(s * _ROWS_PER_TILE, 8)
    pltpu.sync_copy(z_h.at[pl.ds(rbase, _ROWS_PER_TILE)],
                    shared.at[pl.ds(rbase, _ROWS_PER_TILE)])
    plsc.subcore_barrier()
    ebase = (c * _NS + s) * _EPW

    def step(i, carry):
        base = pl.multiple_of(ebase + i * _CH, 8)
        pltpu.sync_copy(di_h.at[pl.ds(base, _CH)], di_v)
        pltpu.sync_copy(d_h.at[pl.ds(base, _CH)], d_v)
        pltpu.sync_copy(d_v, shared.at[di_v], add=True)
        return carry

    lax.fori_loop(0, _NCHUNK, step, 0)
    plsc.subcore_barrier()
    pltpu.sync_copy(shared.at[pl.ds(rbase, _ROWS_PER_TILE)],
                    out_h.at[c, pl.ds(rbase, _ROWS_PER_TILE)])


def _sc_scatter(data, dsti, width):
    zeros = jnp.zeros((_NPAD, width), jnp.float32)
    return pl.kernel(
        _sc_scatter_body,
        out_type=jax.ShapeDtypeStruct((_NC, _NPAD, width), jnp.float32),
        mesh=_mesh(),
        compiler_params=pltpu.CompilerParams(use_tc_tiling_on_sc=False),
        scratch_types=[pltpu.VMEM((_CH,), jnp.int32),
                       pltpu.VMEM((_CH, width), jnp.float32),
                       pltpu.VMEM_SHARED((_NPAD, width), jnp.float32)],
    )(data, dsti, zeros)


def _q_body(h_r, w1, b1, g, be, w2, b2, o_r):
    hdn = jnp.dot(h_r[...], w1[...], preferred_element_type=jnp.float32) + b1[...]
    mu = jnp.mean(hdn, -1, keepdims=True)
    var = jnp.mean(jnp.square(hdn - mu), -1, keepdims=True)
    hdn = (hdn - mu) * lax.rsqrt(var + 1e-5) * g[...] + be[...]
    hdn = jnp.maximum(hdn, 0.0)
    o_r[...] = jnp.dot(hdn, w2[...], preferred_element_type=jnp.float32) + b2[...]


def _qmlp(h, w1, b1, g, be, w2, b2):
    blk = 2000
    grid = _N // blk
    full = lambda shape: pl.BlockSpec(shape, lambda i: (0, 0))
    return pl.pallas_call(
        _q_body,
        grid=(grid,),
        in_specs=[pl.BlockSpec((blk, _IN), lambda i: (i, 0)),
                  full((_IN, _HID)), full((1, _HID)), full((1, _HID)),
                  full((1, _HID)), full((_HID, _OUT)), full((1, _OUT))],
        out_specs=pl.BlockSpec((blk, _OUT), lambda i: (i, 0)),
        out_shape=jax.ShapeDtypeStruct((_N, _OUT), jnp.float32),
    )(h, w1, b1, g, be, w2, b2)


_BE = 2000


def _edge_body(hd_r, qd_r, hs_r, xd_r, xs_r, ea_r, ew_r,
               wk1a, wk1b, wk1cd, bk1, gk, bek, wk2, bk2,
               wv1a, wv1b, wv1cd, bv1, gv, bev, wv2, bv2,
               logits_r, v3_r, gmax_r):
    hi = hd_r[...]
    qd = qd_r[...]
    xd = xd_r[...]
    hj = hs_r[...]
    xs = xs_r[...]

    rel = xd - xs
    spacing = _RMAX / (_NG - 1)
    coeff = -0.5 / (spacing * spacing)
    ones_b = jnp.ones((16, _NG), jnp.float32)
    distb = jnp.sqrt(jnp.dot(rel * rel, ones_b, preferred_element_type=jnp.float32))
    offs = lax.broadcasted_iota(jnp.int32, (1, _NG), 1).astype(jnp.float32) * spacing
    dist_feat = jnp.exp(coeff * jnp.square(distb - offs))

    ea = ea_r[...]
    i0 = lambda s: lax.broadcasted_iota(jnp.int32, s, 0)
    i1 = lambda s: lax.broadcasted_iota(jnp.int32, s, 1)
    exp_ea = (i1((_ED, _ED * _NG)) // _NG == i0((_ED, _ED * _NG))).astype(jnp.float32)
    til_df = (i1((_NG, _ED * _NG)) % _NG == i0((_NG, _ED * _NG))).astype(jnp.float32)
    df = (jnp.dot(ea, exp_ea, preferred_element_type=jnp.float32)
          * jnp.dot(dist_feat, til_df, preferred_element_type=jnp.float32))
    hh = jnp.concatenate([hi, hj], -1)

    def front(w1a, w1b, w1cd, b1, g, be):
        pre = (jnp.dot(ea, w1a[...], preferred_element_type=jnp.float32)
               + jnp.dot(df, w1b[...], preferred_element_type=jnp.float32)
               + jnp.dot(hh, w1cd[...], preferred_element_type=jnp.float32)
               + b1[...])
        mu = jnp.mean(pre, -1, keepdims=True)
        var = jnp.mean(jnp.square(pre - mu), -1, keepdims=True)
        pre = (pre - mu) * lax.rsqrt(var + 1e-5) * g[...] + be[...]
        return jnp.maximum(pre, 0.0)

    hk = front(wk1a, wk1b, wk1cd, bk1, gk, bek)
    k = jnp.dot(hk, wk2[...], preferred_element_type=jnp.float32) + bk2[...]
    hv = front(wv1a, wv1b, wv1cd, bv1, gv, bev)
    v = jnp.dot(hv, wv2[...], preferred_element_type=jnp.float32) + bv2[...]
    v = v * ew_r[...]

    sel = (lax.broadcasted_iota(jnp.int32, (_OUT, _NH), 0) // _DH
           == lax.broadcasted_iota(jnp.int32, (_OUT, _NH), 1)).astype(jnp.float32)
    logits = jnp.dot(k * qd, sel, preferred_element_type=jnp.float32)
    logits = logits * (1.0 / math.sqrt(_DH))
    logits_r[...] = logits

    til_v = (i1((_NH, 3 * _NH)) % _NH == i0((_NH, 3 * _NH))).astype(jnp.float32)
    exp_r = (i1((16, 3 * _NH)) // _NH == i0((16, 3 * _NH))).astype(jnp.float32)
    v3_r[...] = (jnp.dot(v, til_v, preferred_element_type=jnp.float32)
                 * jnp.dot(rel, exp_r, preferred_element_type=jnp.float32))

    @pl.when(pl.program_id(0) == 0)
    def _():
        gmax_r[...] = jnp.full((8, 128), -jnp.inf, jnp.float32)

    cur = gmax_r[0:1, 0:_NH]
    gmax_r[0:1, 0:_NH] = jnp.maximum(cur, jnp.max(logits, 0, keepdims=True))


def _edge_tc(hd, qd, hs, xd, xs, ea, ew, wk, wv):
    grid = _E // _BE
    full = lambda shape: pl.BlockSpec(shape, lambda i: (0, 0))
    wspecs = [full((_ED, _HID)), full((_ED * _NG, _HID)), full((2 * _IN, _HID)),
              full((1, _HID)), full((1, _HID)), full((1, _HID))]
    return pl.pallas_call(
        _edge_body,
        grid=(grid,),
        in_specs=([pl.BlockSpec((_BE, 128), lambda i: (i, 0)),
                   pl.BlockSpec((_BE, 128), lambda i: (i, 0)),
                   pl.BlockSpec((_BE, 128), lambda i: (i, 0)),
                   pl.BlockSpec((_BE, 16), lambda i: (i, 0)),
                   pl.BlockSpec((_BE, 16), lambda i: (i, 0)),
                   pl.BlockSpec((_BE, _ED), lambda i: (i, 0)),
                   pl.BlockSpec((_BE, 1), lambda i: (i, 0))]
                  + wspecs + [full((_HID, _OUT)), full((1, _OUT))]
                  + wspecs + [full((_HID, _NH)), full((1, _NH))]),
        out_specs=[pl.BlockSpec((_BE, _NH), lambda i: (i, 0)),
                   pl.BlockSpec((_BE, 3 * _NH), lambda i: (i, 0)),
                   pl.BlockSpec((8, 128), lambda i: (0, 0))],
        out_shape=[jax.ShapeDtypeStruct((_E, _NH), jnp.float32),
                   jax.ShapeDtypeStruct((_E, 3 * _NH), jnp.float32),
                   jax.ShapeDtypeStruct((8, 128), jnp.float32)],
    )(hd, qd, hs, xd, xs, ea, ew, *wk, *wv)


def _exp_body(l_r, gmax_r, o_r):
    o_r[...] = jnp.exp(l_r[...] - gmax_r[0:1, 0:_NH])


def _exp_tc(logits, gmax):
    blk = 8000
    return pl.pallas_call(
        _exp_body,
        grid=(_E // blk,),
        in_specs=[pl.BlockSpec((blk, _NH), lambda i: (i, 0)),
                  pl.BlockSpec((8, 128), lambda i: (0, 0))],
        out_specs=pl.BlockSpec((blk, _NH), lambda i: (i, 0)),
        out_shape=jax.ShapeDtypeStruct((_E, _NH), jnp.float32),
    )(logits, gmax)


def _av_body(ex_r, dg_r, v3_r, o_r):
    al = ex_r[...] / (dg_r[...] + 1e-16)
    i0 = lambda s: lax.broadcasted_iota(jnp.int32, s, 0)
    i1 = lambda s: lax.broadcasted_iota(jnp.int32, s, 1)
    til = (i1((_NH, 3 * _NH)) % _NH == i0((_NH, 3 * _NH))).astype(jnp.float32)
    o_r[...] = jnp.dot(al, til, preferred_element_type=jnp.float32) * v3_r[...]


def _av_tc(ex, dg, v3):
    blk = 8000
    return pl.pallas_call(
        _av_body,
        grid=(_E // blk,),
        in_specs=[pl.BlockSpec((blk, _NH), lambda i: (i, 0)),
                  pl.BlockSpec((blk, _NH), lambda i: (i, 0)),
                  pl.BlockSpec((blk, 3 * _NH), lambda i: (i, 0))],
        out_specs=pl.BlockSpec((blk, 3 * _NH), lambda i: (i, 0)),
        out_shape=jax.ShapeDtypeStruct((_E, 3 * _NH), jnp.float32),
    )(ex, dg, v3)


def kernel(x, h, edge_attr, e_w, xk_w1, xk_b1, xk_g, xk_be, xk_w2, xk_b2,
           xv_w1, xv_b1, xv_g, xv_be, xv_w2, xv_b2,
           xq_w1, xq_b1, xq_g, xq_be, xq_w2, xq_b2, edge_index):
    src = edge_index[0]
    dst = edge_index[1]
    row = lambda a: a.reshape(1, -1)

    tq = _qmlp(h, xq_w1, row(xq_b1), row(xq_g), row(xq_be), xq_w2, row(xq_b2))
    tx = jnp.pad(x, ((0, 0), (0, 13)))
    hd, qd, hs, xd, xs = _sc_gather5(h, tq, tx, dst, src)

    def split(w1):
        return w1[0:_ED], w1[_ED:_ED + _ED * _NG], w1[_ED + _ED * _NG:]

    ka, kb, kcd = split(xk_w1)
    va, vb, vcd = split(xv_w1)
    wk = (ka, kb, kcd, row(xk_b1), row(xk_g), row(xk_be), xk_w2, row(xk_b2))
    wv = (va, vb, vcd, row(xv_b1), row(xv_g), row(xv_be), xv_w2, row(xv_b2))

    logits, v3, gmax = _edge_tc(hd, qd, hs, xd, xs, edge_attr, e_w, wk, wv)
    ex = _exp_tc(logits, gmax)

    dpart = _sc_scatter(ex, dst, _NH)
    denom = dpart[0] + dpart[1]
    dg = _sc_gather1(denom, dst, _NH)

    m3 = _av_tc(ex, dg, v3)
    opart = _sc_scatter(m3, dst, 3 * _NH)
    out = (opart[0, :_N] + opart[1, :_N]).reshape(_N, 3, _NH)
    return jnp.mean(out, -1)


# speedup vs baseline: 20.2247x; 1.1589x over previous
import functools
import math

import jax
import jax.numpy as jnp
from jax import lax
from jax.experimental import pallas as pl
from jax.experimental.pallas import tpu as pltpu
from jax.experimental.pallas import tpu_sc as plsc

_N = 10000
_E = 320000
_IN = 128
_HID = 128
_OUT = 128
_NH = 16
_DH = 8
_ED = 4
_NG = 20
_RMAX = 10.0

_NC = 2
_NS = 16
_NW = _NC * _NS
_EPW = _E // _NW
_CH = 80
_NCHUNK = _EPW // _CH
_NPAD = 10240
_ROWS_PER_TILE = _NPAD // _NS

_DST_W = _IN + _OUT + 16
_SRC_W = _IN + 16


def _mesh():
    return plsc.VectorSubcoreMesh(core_axis_name="c", subcore_axis_name="s")


def _sc_gather5_body(th_h, tq_h, tx_h, di_h, si_h,
                     hd_h, qd_h, hs_h, xd_h, xs_h,
                     di_v, si_v, bhd, bqd, bhs, bxd, bxs,
                     s1, s2, s3, s4, s5):
    c = lax.axis_index("c")
    s = lax.axis_index("s")
    ebase = (c * _NS + s) * _EPW

    def step(i, carry):
        base = pl.multiple_of(ebase + i * _CH, 8)
        pltpu.sync_copy(di_h.at[pl.ds(base, _CH)], di_v)
        pltpu.sync_copy(si_h.at[pl.ds(base, _CH)], si_v)
        cps = [pltpu.async_copy(th_h.at[di_v], bhd, s1),
               pltpu.async_copy(tq_h.at[di_v], bqd, s2),
               pltpu.async_copy(th_h.at[si_v], bhs, s3),
               pltpu.async_copy(tx_h.at[di_v], bxd, s4),
               pltpu.async_copy(tx_h.at[si_v], bxs, s5)]
        for cp in cps:
            cp.wait()
        pltpu.sync_copy(bhd, hd_h.at[pl.ds(base, _CH)])
        pltpu.sync_copy(bqd, qd_h.at[pl.ds(base, _CH)])
        pltpu.sync_copy(bhs, hs_h.at[pl.ds(base, _CH)])
        pltpu.sync_copy(bxd, xd_h.at[pl.ds(base, _CH)])
        pltpu.sync_copy(bxs, xs_h.at[pl.ds(base, _CH)])
        return carry

    lax.fori_loop(0, _NCHUNK, step, 0)


def _sc_gather5(th, tq, tx, dsti, srci):
    f32 = jnp.float32
    return pl.kernel(
        _sc_gather5_body,
        out_type=[jax.ShapeDtypeStruct((_E, 128), f32),
                  jax.ShapeDtypeStruct((_E, 128), f32),
                  jax.ShapeDtypeStruct((_E, 128), f32),
                  jax.ShapeDtypeStruct((_E, 16), f32),
                  jax.ShapeDtypeStruct((_E, 16), f32)],
        mesh=_mesh(),
        compiler_params=pltpu.CompilerParams(use_tc_tiling_on_sc=False),
        scratch_types=[pltpu.VMEM((_CH,), jnp.int32),
                       pltpu.VMEM((_CH,), jnp.int32),
                       pltpu.VMEM((_CH, 128), f32),
                       pltpu.VMEM((_CH, 128), f32),
                       pltpu.VMEM((_CH, 128), f32),
                       pltpu.VMEM((_CH, 16), f32),
                       pltpu.VMEM((_CH, 16), f32),
                       pltpu.SemaphoreType.DMA,
                       pltpu.SemaphoreType.DMA,
                       pltpu.SemaphoreType.DMA,
                       pltpu.SemaphoreType.DMA,
                       pltpu.SemaphoreType.DMA],
    )(th, tq, tx, dsti, srci)


def _sc_gather1_body(tab_h, di_h, o_h, di_v, b_v, sem):
    c = lax.axis_index("c")
    s = lax.axis_index("s")
    ebase = (c * _NS + s) * _EPW

    def step(i, carry):
        base = pl.multiple_of(ebase + i * _CH, 8)
        pltpu.sync_copy(di_h.at[pl.ds(base, _CH)], di_v)
        pltpu.async_copy(tab_h.at[di_v], b_v, sem).wait()
        pltpu.sync_copy(b_v, o_h.at[pl.ds(base, _CH)])
        return carry

    lax.fori_loop(0, _NCHUNK, step, 0)


def _sc_gather1(tab, dsti, width):
    return pl.kernel(
        _sc_gather1_body,
        out_type=jax.ShapeDtypeStruct((_E, width), jnp.float32),
        mesh=_mesh(),
        compiler_params=pltpu.CompilerParams(use_tc_tiling_on_sc=False),
        scratch_types=[pltpu.VMEM((_CH,), jnp.int32),
                       pltpu.VMEM((_CH, width), jnp.float32),
                       pltpu.SemaphoreType.DMA],
    )(tab, dsti)


def _sc_scatter_body(d_h, di_h, z_h, out_h, di_v, d_v, shared):
    c = lax.axis_index("c")
    s = lax.axis_index("s")
    rbase = pl.multiple_of(s * _ROWS_PER_TILE, 8)
    pltpu.sync_copy(z_h.at[pl.ds(rbase, _ROWS_PER_TILE)],
                    shared.at[pl.ds(rbase, _ROWS_PER_TILE)])
    plsc.subcore_barrier()
    ebase = (c * _NS + s) * _EPW

    def step(i, carry):
        base = pl.multiple_of(ebase + i * _CH, 8)
        pltpu.sync_copy(di_h.at[pl.ds(base, _CH)], di_v)
        pltpu.sync_copy(d_h.at[pl.ds(base, _CH)], d_v)
        pltpu.sync_copy(d_v, shared.at[di_v], add=True)
        return carry

    lax.fori_loop(0, _NCHUNK, step, 0)
    plsc.subcore_barrier()
    pltpu.sync_copy(shared.at[pl.ds(rbase, _ROWS_PER_TILE)],
                    out_h.at[c, pl.ds(rbase, _ROWS_PER_TILE)])


def _sc_scatter(data, dsti, width):
    zeros = jnp.zeros((_NPAD, width), jnp.float32)
    return pl.kernel(
        _sc_scatter_body,
        out_type=jax.ShapeDtypeStruct((_NC, _NPAD, width), jnp.float32),
        mesh=_mesh(),
        compiler_params=pltpu.CompilerParams(use_tc_tiling_on_sc=False),
        scratch_types=[pltpu.VMEM((_CH,), jnp.int32),
                       pltpu.VMEM((_CH, width), jnp.float32),
                       pltpu.VMEM_SHARED((_NPAD, width), jnp.float32)],
    )(data, dsti, zeros)


def _sc_av_scatter_body(den_h, ex_h, v3_h, di_h, z_h, out_h,
                        di_v, dg_v, ex_v, v3_v, shared, sem):
    c = lax.axis_index("c")
    s = lax.axis_index("s")
    rbase = pl.multiple_of(s * _ROWS_PER_TILE, 8)
    pltpu.sync_copy(z_h.at[pl.ds(rbase, _ROWS_PER_TILE)],
                    shared.at[pl.ds(rbase, _ROWS_PER_TILE)])
    plsc.subcore_barrier()
    ebase = (c * _NS + s) * _EPW

    def step(i, carry):
        base = pl.multiple_of(ebase + i * _CH, 8)
        pltpu.sync_copy(di_h.at[pl.ds(base, _CH)], di_v)
        cp = pltpu.async_copy(den_h.at[di_v], dg_v, sem)
        pltpu.sync_copy(ex_h.at[pl.ds(base, _CH)], ex_v)
        pltpu.sync_copy(v3_h.at[pl.ds(base, _CH)], v3_v)
        cp.wait()

        def row4(r4, carry2):
            for u in range(4):
                r = r4 * 4 + u
                al = ex_v[r, :] / (dg_v[r, :] + 1e-16)
                for cc in range(3):
                    sl = pl.ds(cc * _NH, _NH)
                    v3_v[r, sl] = al * v3_v[r, sl]
            return carry2

        lax.fori_loop(0, _CH // 4, row4, 0)
        pltpu.sync_copy(v3_v, shared.at[di_v], add=True)
        return carry

    lax.fori_loop(0, _NCHUNK, step, 0)
    plsc.subcore_barrier()
    pltpu.sync_copy(shared.at[pl.ds(rbase, _ROWS_PER_TILE)],
                    out_h.at[c, pl.ds(rbase, _ROWS_PER_TILE)])


def _sc_av_scatter(denom, ex, v3, dsti):
    f32 = jnp.float32
    zeros = jnp.zeros((_NPAD, 3 * _NH), f32)
    return pl.kernel(
        _sc_av_scatter_body,
        out_type=jax.ShapeDtypeStruct((_NC, _NPAD, 3 * _NH), f32),
        mesh=_mesh(),
        compiler_params=pltpu.CompilerParams(use_tc_tiling_on_sc=False),
        scratch_types=[pltpu.VMEM((_CH,), jnp.int32),
                       pltpu.VMEM((_CH, _NH), f32),
                       pltpu.VMEM((_CH, _NH), f32),
                       pltpu.VMEM((_CH, 3 * _NH), f32),
                       pltpu.VMEM_SHARED((_NPAD, 3 * _NH), f32),
                       pltpu.SemaphoreType.DMA],
    )(denom, ex, v3, dsti, zeros)


def _q_body(h_r, w1, b1, g, be, w2, b2, o_r):
    hdn = jnp.dot(h_r[...], w1[...], preferred_element_type=jnp.float32) + b1[...]
    mu = jnp.mean(hdn, -1, keepdims=True)
    var = jnp.mean(jnp.square(hdn - mu), -1, keepdims=True)
    hdn = (hdn - mu) * lax.rsqrt(var + 1e-5) * g[...] + be[...]
    hdn = jnp.maximum(hdn, 0.0)
    o_r[...] = jnp.dot(hdn, w2[...], preferred_element_type=jnp.float32) + b2[...]


def _qmlp(h, w1, b1, g, be, w2, b2):
    blk = 2000
    grid = _N // blk
    full = lambda shape: pl.BlockSpec(shape, lambda i: (0, 0))
    return pl.pallas_call(
        _q_body,
        grid=(grid,),
        in_specs=[pl.BlockSpec((blk, _IN), lambda i: (i, 0)),
                  full((_IN, _HID)), full((1, _HID)), full((1, _HID)),
                  full((1, _HID)), full((_HID, _OUT)), full((1, _OUT))],
        out_specs=pl.BlockSpec((blk, _OUT), lambda i: (i, 0)),
        out_shape=jax.ShapeDtypeStruct((_N, _OUT), jnp.float32),
    )(h, w1, b1, g, be, w2, b2)


_BE = 2000


def _edge_body(hd_r, qd_r, hs_r, xd_r, xs_r, ea_r, ew_r,
               wk1a, wk1b, wk1cd, bk1, gk, bek, wk2, bk2,
               wv1a, wv1b, wv1cd, bv1, gv, bev, wv2, bv2,
               logits_r, v3_r, gmax_r):
    hi = hd_r[...]
    qd = qd_r[...]
    xd = xd_r[...]
    hj = hs_r[...]
    xs = xs_r[...]

    rel = xd - xs
    spacing = _RMAX / (_NG - 1)
    coeff = -0.5 / (spacing * spacing)
    ones_b = jnp.ones((16, _NG), jnp.float32)
    distb = jnp.sqrt(jnp.dot(rel * rel, ones_b, preferred_element_type=jnp.float32))
    offs = lax.broadcasted_iota(jnp.int32, (1, _NG), 1).astype(jnp.float32) * spacing
    dist_feat = jnp.exp(coeff * jnp.square(distb - offs))

    ea = ea_r[...]
    i0 = lambda s: lax.broadcasted_iota(jnp.int32, s, 0)
    i1 = lambda s: lax.broadcasted_iota(jnp.int32, s, 1)
    exp_ea = (i1((_ED, _ED * _NG)) // _NG == i0((_ED, _ED * _NG))).astype(jnp.float32)
    til_df = (i1((_NG, _ED * _NG)) % _NG == i0((_NG, _ED * _NG))).astype(jnp.float32)
    df = (jnp.dot(ea, exp_ea, preferred_element_type=jnp.float32)
          * jnp.dot(dist_feat, til_df, preferred_element_type=jnp.float32))
    hh = jnp.concatenate([hi, hj], -1)

    def front(w1a, w1b, w1cd, b1, g, be):
        pre = (jnp.dot(ea, w1a[...], preferred_element_type=jnp.float32)
               + jnp.dot(df, w1b[...], preferred_element_type=jnp.float32)
               + jnp.dot(hh, w1cd[...], preferred_element_type=jnp.float32)
               + b1[...])
        mu = jnp.mean(pre, -1, keepdims=True)
        var = jnp.mean(jnp.square(pre - mu), -1, keepdims=True)
        pre = (pre - mu) * lax.rsqrt(var + 1e-5) * g[...] + be[...]
        return jnp.maximum(pre, 0.0)

    hk = front(wk1a, wk1b, wk1cd, bk1, gk, bek)
    k = jnp.dot(hk, wk2[...], preferred_element_type=jnp.float32) + bk2[...]
    hv = front(wv1a, wv1b, wv1cd, bv1, gv, bev)
    v = jnp.dot(hv, wv2[...], preferred_element_type=jnp.float32) + bv2[...]
    v = v * ew_r[...]

    sel = (lax.broadcasted_iota(jnp.int32, (_OUT, _NH), 0) // _DH
           == lax.broadcasted_iota(jnp.int32, (_OUT, _NH), 1)).astype(jnp.float32)
    logits = jnp.dot(k * qd, sel, preferred_element_type=jnp.float32)
    logits = logits * (1.0 / math.sqrt(_DH))
    logits_r[...] = logits

    til_v = (i1((_NH, 3 * _NH)) % _NH == i0((_NH, 3 * _NH))).astype(jnp.float32)
    exp_r = (i1((16, 3 * _NH)) // _NH == i0((16, 3 * _NH))).astype(jnp.float32)
    v3_r[...] = (jnp.dot(v, til_v, preferred_element_type=jnp.float32)
                 * jnp.dot(rel, exp_r, preferred_element_type=jnp.float32))

    @pl.when(pl.program_id(0) == 0)
    def _():
        gmax_r[...] = jnp.full((8, 128), -jnp.inf, jnp.float32)

    cur = gmax_r[0:1, 0:_NH]
    gmax_r[0:1, 0:_NH] = jnp.maximum(cur, jnp.max(logits, 0, keepdims=True))


def _edge_tc(hd, qd, hs, xd, xs, ea, ew, wk, wv):
    grid = _E // _BE
    full = lambda shape: pl.BlockSpec(shape, lambda i: (0, 0))
    wspecs = [full((_ED, _HID)), full((_ED * _NG, _HID)), full((2 * _IN, _HID)),
              full((1, _HID)), full((1, _HID)), full((1, _HID))]
    return pl.pallas_call(
        _edge_body,
        grid=(grid,),
        in_specs=([pl.BlockSpec((_BE, 128), lambda i: (i, 0)),
                   pl.BlockSpec((_BE, 128), lambda i: (i, 0)),
                   pl.BlockSpec((_BE, 128), lambda i: (i, 0)),
                   pl.BlockSpec((_BE, 16), lambda i: (i, 0)),
                   pl.BlockSpec((_BE, 16), lambda i: (i, 0)),
                   pl.BlockSpec((_BE, _ED), lambda i: (i, 0)),
                   pl.BlockSpec((_BE, 1), lambda i: (i, 0))]
                  + wspecs + [full((_HID, _OUT)), full((1, _OUT))]
                  + wspecs + [full((_HID, _NH)), full((1, _NH))]),
        out_specs=[pl.BlockSpec((_BE, _NH), lambda i: (i, 0)),
                   pl.BlockSpec((_BE, 3 * _NH), lambda i: (i, 0)),
                   pl.BlockSpec((8, 128), lambda i: (0, 0))],
        out_shape=[jax.ShapeDtypeStruct((_E, _NH), jnp.float32),
                   jax.ShapeDtypeStruct((_E, 3 * _NH), jnp.float32),
                   jax.ShapeDtypeStruct((8, 128), jnp.float32)],
    )(hd, qd, hs, xd, xs, ea, ew, *wk, *wv)


def _exp_body(l_r, gmax_r, o_r):
    o_r[...] = jnp.exp(l_r[...] - gmax_r[0:1, 0:_NH])


def _exp_tc(logits, gmax):
    blk = 8000
    return pl.pallas_call(
        _exp_body,
        grid=(_E // blk,),
        in_specs=[pl.BlockSpec((blk, _NH), lambda i: (i, 0)),
                  pl.BlockSpec((8, 128), lambda i: (0, 0))],
        out_specs=pl.BlockSpec((blk, _NH), lambda i: (i, 0)),
        out_shape=jax.ShapeDtypeStruct((_E, _NH), jnp.float32),
    )(logits, gmax)


def _av_body(ex_r, dg_r, v3_r, o_r):
    al = ex_r[...] / (dg_r[...] + 1e-16)
    i0 = lambda s: lax.broadcasted_iota(jnp.int32, s, 0)
    i1 = lambda s: lax.broadcasted_iota(jnp.int32, s, 1)
    til = (i1((_NH, 3 * _NH)) % _NH == i0((_NH, 3 * _NH))).astype(jnp.float32)
    o_r[...] = jnp.dot(al, til, preferred_element_type=jnp.float32) * v3_r[...]


def _av_tc(ex, dg, v3):
    blk = 8000
    return pl.pallas_call(
        _av_body,
        grid=(_E // blk,),
        in_specs=[pl.BlockSpec((blk, _NH), lambda i: (i, 0)),
                  pl.BlockSpec((blk, _NH), lambda i: (i, 0)),
                  pl.BlockSpec((blk, 3 * _NH), lambda i: (i, 0))],
        out_specs=pl.BlockSpec((blk, 3 * _NH), lambda i: (i, 0)),
        out_shape=jax.ShapeDtypeStruct((_E, 3 * _NH), jnp.float32),
    )(ex, dg, v3)


def kernel(x, h, edge_attr, e_w, xk_w1, xk_b1, xk_g, xk_be, xk_w2, xk_b2,
           xv_w1, xv_b1, xv_g, xv_be, xv_w2, xv_b2,
           xq_w1, xq_b1, xq_g, xq_be, xq_w2, xq_b2, edge_index):
    src = edge_index[0]
    dst = edge_index[1]
    row = lambda a: a.reshape(1, -1)

    tq = _qmlp(h, xq_w1, row(xq_b1), row(xq_g), row(xq_be), xq_w2, row(xq_b2))
    tx = jnp.pad(x, ((0, 0), (0, 13)))
    hd, qd, hs, xd, xs = _sc_gather5(h, tq, tx, dst, src)

    def split(w1):
        return w1[0:_ED], w1[_ED:_ED + _ED * _NG], w1[_ED + _ED * _NG:]

    ka, kb, kcd = split(xk_w1)
    va, vb, vcd = split(xv_w1)
    wk = (ka, kb, kcd, row(xk_b1), row(xk_g), row(xk_be), xk_w2, row(xk_b2))
    wv = (va, vb, vcd, row(xv_b1), row(xv_g), row(xv_be), xv_w2, row(xv_b2))

    logits, v3, gmax = _edge_tc(hd, qd, hs, xd, xs, edge_attr, e_w, wk, wv)
    ex = _exp_tc(logits, gmax)

    dpart = _sc_scatter(ex, dst, _NH)
    denom = dpart[0] + dpart[1]
    opart = _sc_av_scatter(denom, ex, v3, dst)
    out = (opart[0, :_N] + opart[1, :_N]).reshape(_N, 3, _NH)
    return jnp.mean(out, -1)
